# baseline (speedup 1.0000x reference)
import jax
import jax.numpy as jnp
from jax.experimental import pallas as pl

SEQ = 200
DIM = 64
FEAT = SEQ * DIM


def _add_kernel(ex_ref, cat_ref, pos_ref, out_ref):
    out_ref[:] = ex_ref[:] + cat_ref[:] + pos_ref[:]


def kernel(exercises, categories, position_embed):
    B = exercises.shape[0]
    ex2 = exercises.reshape(B, FEAT)
    cat2 = categories.reshape(B, FEAT)
    pos2 = position_embed.reshape(1, FEAT)
    BB = 64
    out = pl.pallas_call(
        _add_kernel,
        grid=(B // BB,),
        in_specs=[
            pl.BlockSpec((BB, FEAT), lambda i: (i, 0)),
            pl.BlockSpec((BB, FEAT), lambda i: (i, 0)),
            pl.BlockSpec((1, FEAT), lambda i: (0, 0)),
        ],
        out_specs=pl.BlockSpec((BB, FEAT), lambda i: (i, 0)),
        out_shape=jax.ShapeDtypeStruct((B, FEAT), jnp.float32),
    )(ex2, cat2, pos2)
    return out.reshape(B, SEQ, DIM)
